# baseline (speedup 1.0000x reference)
import functools

import jax
import jax.numpy as jnp
from jax.experimental import pallas as pl
from jax.experimental.pallas import tpu as pltpu

_NCHUNK = 4


def _kmeans_loss_body(x_ref, mu_hbm, o_ref, lhs_ref, rhs_ref, sem_m,
                      *, inv_g):
    G, K = mu_hbm.shape
    kc = K // _NCHUNK

    cp_m = [
        pltpu.make_async_copy(mu_hbm.at[:, j * kc:(j + 1) * kc],
                              lhs_ref.at[0:G, j * kc:(j + 1) * kc],
                              sem_m.at[j])
        for j in range(_NCHUNK)
    ]
    for cp in cp_m:
        cp.start()

    x = x_ref[...]
    xsq = jnp.sum(x * x, axis=1, keepdims=True)
    rhs_ref[:, 0:G] = x
    rhs_ref[:, G:G + 1] = jnp.ones((x.shape[0], 1), jnp.float32)
    rhs_ref[:, G + 1:G + 2] = -0.5 * xsq

    run = None
    for j in range(_NCHUNK):
        cp_m[j].wait()
        js = slice(j * kc, (j + 1) * kc)
        mj = lhs_ref[0:G, js]
        musq = jnp.sum(mj * mj, axis=0, keepdims=True)
        lhs_ref[G:G + 1, js] = -0.5 * musq
        lhs_ref[G + 1:G + 2, js] = jnp.ones_like(musq)
        p = jax.lax.dot_general(
            lhs_ref[:, js], rhs_ref[...], (((0,), (1,)), ((), ())),
            preferred_element_type=jnp.float32)
        pmax = jnp.max(p, axis=0, keepdims=True)
        run = pmax if run is None else jnp.maximum(run, pmax)
    o_ref[...] = run * (-2.0 * inv_g)


def kernel(images, mu):
    B, G = images.shape
    _, K = mu.shape
    out = pl.pallas_call(
        functools.partial(_kmeans_loss_body, inv_g=1.0 / G),
        out_shape=jax.ShapeDtypeStruct((1, B), jnp.float32),
        grid=(1,),
        in_specs=[
            pl.BlockSpec((B, G), lambda i: (0, 0)),
            pl.BlockSpec(memory_space=pl.ANY),
        ],
        out_specs=pl.BlockSpec((1, B), lambda i: (0, 0)),
        scratch_shapes=[
            pltpu.VMEM((G + 2, K), jnp.float32),
            pltpu.VMEM((B, G + 2), jnp.float32),
            pltpu.SemaphoreType.DMA((_NCHUNK,)),
        ],
    )(images, mu)
    return out.reshape(B)
